# baseline (speedup 1.0000x reference)
import jax
import jax.numpy as jnp
from jax.experimental import pallas as pl
from jax.experimental.pallas import tpu as pltpu

_B, _N, _F, _CH = 32, 200, 128, 32


def _elu(v):
    return jnp.where(v > 0, v, jnp.exp(jnp.minimum(v, 0.0)) - 1.0)
_H1, _H2 = 512, 256
_KCHUNK = 640


def _gcn_kernel(x_ref, a_ref, w1_ref, b1_ref, w2_ref, b2_ref, o_ref):
    xb = x_ref[0]
    a = a_ref[...]
    xw = jnp.dot(xb, w1_ref[...], preferred_element_type=jnp.float32)
    h = jnp.dot(a, xw, preferred_element_type=jnp.float32) + b1_ref[...]
    h = _elu(h)
    hw = jnp.dot(h, w2_ref[...], preferred_element_type=jnp.float32)
    h2 = jnp.dot(a, hw, preferred_element_type=jnp.float32) + b2_ref[...]
    o_ref[0] = _elu(h2)


def _mlp_kernel(f_ref, wf1_ref, bf1_ref, wf2_ref, bf2_ref, wf3_ref, bf3_ref,
                o_ref, acc_ref):
    k = pl.program_id(0)

    @pl.when(k == 0)
    def _init():
        acc_ref[...] = jnp.zeros_like(acc_ref)

    acc_ref[...] += jnp.dot(f_ref[...], wf1_ref[...],
                            preferred_element_type=jnp.float32)

    @pl.when(k == pl.num_programs(0) - 1)
    def _tail():
        h = jax.nn.relu(acc_ref[...] + bf1_ref[...])
        h = jax.nn.relu(jnp.dot(h, wf2_ref[...],
                                preferred_element_type=jnp.float32)
                        + bf2_ref[...])
        o_ref[...] = jax.nn.sigmoid(
            jnp.dot(h, wf3_ref[...], preferred_element_type=jnp.float32)
            + bf3_ref[...])


def kernel(x, a, W1, b1, W2, b2, Wf1, bf1, Wf2, bf2, Wf3, bf3):
    h2 = pl.pallas_call(
        _gcn_kernel,
        grid=(_B,),
        in_specs=[
            pl.BlockSpec((1, _N, _F), lambda b: (b, 0, 0)),
            pl.BlockSpec((_N, _N), lambda b: (0, 0)),
            pl.BlockSpec((_F, _CH), lambda b: (0, 0)),
            pl.BlockSpec((1, _CH), lambda b: (0, 0)),
            pl.BlockSpec((_CH, _CH), lambda b: (0, 0)),
            pl.BlockSpec((1, _CH), lambda b: (0, 0)),
        ],
        out_specs=pl.BlockSpec((1, _N, _CH), lambda b: (b, 0, 0)),
        out_shape=jax.ShapeDtypeStruct((_B, _N, _CH), jnp.float32),
        compiler_params=pltpu.CompilerParams(
            dimension_semantics=("parallel",)),
    )(x, a, W1, b1.reshape(1, _CH), W2, b2.reshape(1, _CH))

    flat = h2.reshape(_B, _N * _CH)
    nk = (_N * _CH) // _KCHUNK

    out = pl.pallas_call(
        _mlp_kernel,
        grid=(nk,),
        in_specs=[
            pl.BlockSpec((_B, _KCHUNK), lambda k: (0, k)),
            pl.BlockSpec((_KCHUNK, _H1), lambda k: (k, 0)),
            pl.BlockSpec((1, _H1), lambda k: (0, 0)),
            pl.BlockSpec((_H1, _H2), lambda k: (0, 0)),
            pl.BlockSpec((1, _H2), lambda k: (0, 0)),
            pl.BlockSpec((_H2, 1), lambda k: (0, 0)),
            pl.BlockSpec((1, 1), lambda k: (0, 0)),
        ],
        out_specs=pl.BlockSpec((_B, 1), lambda k: (0, 0)),
        out_shape=jax.ShapeDtypeStruct((_B, 1), jnp.float32),
        scratch_shapes=[pltpu.VMEM((_B, _H1), jnp.float32)],
        compiler_params=pltpu.CompilerParams(
            dimension_semantics=("arbitrary",)),
    )(flat, Wf1, bf1.reshape(1, _H1), Wf2, bf2.reshape(1, _H2),
      Wf3, bf3.reshape(1, 1))
    return out


# speedup vs baseline: 2.9152x; 2.9152x over previous
import jax
import jax.numpy as jnp
from jax.experimental import pallas as pl
from jax.experimental.pallas import tpu as pltpu

_B, _N, _F, _CH = 32, 200, 128, 32
_H1, _H2 = 512, 256
_NK = 4
_KC = (_N * _CH) // _NK


def _elu(v):
    return jnp.where(v > 0, v, jnp.exp(jnp.minimum(v, 0.0)) - 1.0)


def _net_kernel(x_ref, a_ref, w1_ref, b1t_ref, w2_ref, b2_ref,
                wf1_hbm, bf1_ref, wf2_ref, bf2_ref, wf3_ref, bf3_ref,
                o_ref, wf1_vmem, sems):
    for k in range(_NK):
        pltpu.make_async_copy(
            wf1_hbm.at[k * _KC:(k + 1) * _KC, :],
            wf1_vmem.at[k], sems.at[k]).start()

    xw = jnp.dot(x_ref[...], w1_ref[...],
                 preferred_element_type=jnp.float32)
    y = jnp.concatenate(
        [xw[b * _N:(b + 1) * _N, :] for b in range(_B)], axis=1)
    a = a_ref[...]
    h1 = _elu(jnp.dot(a, y, preferred_element_type=jnp.float32)
              + b1t_ref[...])
    u = jnp.dot(a, h1, preferred_element_type=jnp.float32)
    v = jnp.concatenate(
        [u[:, b * _CH:(b + 1) * _CH] for b in range(_B)], axis=0)
    h2 = _elu(jnp.dot(v, w2_ref[...], preferred_element_type=jnp.float32)
              + b2_ref[...])

    s3 = h2.reshape(_B, _N, _CH)
    t1 = jnp.transpose(s3, (1, 0, 2))
    t2 = jnp.transpose(t1, (0, 2, 1))
    flat = t2.reshape(_N * _CH, _B).T

    acc = jnp.zeros((_B, _H1), jnp.float32)
    for k in range(_NK):
        pltpu.make_async_copy(
            wf1_hbm.at[k * _KC:(k + 1) * _KC, :],
            wf1_vmem.at[k], sems.at[k]).wait()
        acc = acc + jnp.dot(flat[:, k * _KC:(k + 1) * _KC], wf1_vmem[k],
                            preferred_element_type=jnp.float32)

    t = jax.nn.relu(acc + bf1_ref[...])
    t = jax.nn.relu(jnp.dot(t, wf2_ref[...],
                            preferred_element_type=jnp.float32) + bf2_ref[...])
    o_ref[...] = jax.nn.sigmoid(
        jnp.dot(t, wf3_ref[...], preferred_element_type=jnp.float32)
        + bf3_ref[...])


def kernel(x, a, W1, b1, W2, b2, Wf1, bf1, Wf2, bf2, Wf3, bf3):
    x2 = x.reshape(_B * _N, _F)
    b1t = jnp.tile(b1, _B).reshape(1, _B * _CH)

    out = pl.pallas_call(
        _net_kernel,
        in_specs=[
            pl.BlockSpec((_B * _N, _F), lambda: (0, 0)),
            pl.BlockSpec((_N, _N), lambda: (0, 0)),
            pl.BlockSpec((_F, _CH), lambda: (0, 0)),
            pl.BlockSpec((1, _B * _CH), lambda: (0, 0)),
            pl.BlockSpec((_CH, _CH), lambda: (0, 0)),
            pl.BlockSpec((1, _CH), lambda: (0, 0)),
            pl.BlockSpec(memory_space=pl.ANY),
            pl.BlockSpec((1, _H1), lambda: (0, 0)),
            pl.BlockSpec((_H1, _H2), lambda: (0, 0)),
            pl.BlockSpec((1, _H2), lambda: (0, 0)),
            pl.BlockSpec((_H2, 1), lambda: (0, 0)),
            pl.BlockSpec((1, 1), lambda: (0, 0)),
        ],
        out_specs=pl.BlockSpec((_B, 1), lambda: (0, 0)),
        out_shape=jax.ShapeDtypeStruct((_B, 1), jnp.float32),
        scratch_shapes=[
            pltpu.VMEM((_NK, _KC, _H1), jnp.float32),
            pltpu.SemaphoreType.DMA((_NK,)),
        ],
    )(x2, a, W1, b1t, W2, b2.reshape(1, _CH), Wf1,
      bf1.reshape(1, _H1), Wf2, bf2.reshape(1, _H2), Wf3, bf3.reshape(1, 1))
    return out


# speedup vs baseline: 2.9243x; 1.0031x over previous
import jax
import jax.numpy as jnp
from jax.experimental import pallas as pl
from jax.experimental.pallas import tpu as pltpu

_B, _N, _F, _CH = 32, 200, 128, 32
_H1, _H2 = 512, 256
_NK = 4
_KC = (_N * _CH) // _NK


def _elu(v):
    return jnp.where(v > 0, v, jnp.exp(jnp.minimum(v, 0.0)) - 1.0)


def _net_kernel(x_ref, a_ref, w1_ref, b1t_ref, w2_ref, b2_ref,
                wf1_hbm, bf1_ref, wf2_ref, bf2_ref, wf3_ref, bf3_ref,
                o_ref, wf1_vmem, sems):
    for k in range(_NK):
        pltpu.make_async_copy(
            wf1_hbm.at[k * _KC:(k + 1) * _KC, :],
            wf1_vmem.at[k], sems.at[k]).start()

    bf = jnp.bfloat16
    xw = jnp.dot(x_ref[...].astype(bf), w1_ref[...].astype(bf),
                 preferred_element_type=jnp.float32)
    y = jnp.concatenate(
        [xw[b * _N:(b + 1) * _N, :] for b in range(_B)], axis=1)
    a = a_ref[...].astype(bf)
    h1 = _elu(jnp.dot(a, y.astype(bf), preferred_element_type=jnp.float32)
              + b1t_ref[...])
    u = jnp.dot(a, h1.astype(bf), preferred_element_type=jnp.float32)
    v = jnp.concatenate(
        [u[:, b * _CH:(b + 1) * _CH] for b in range(_B)], axis=0)
    h2 = _elu(jnp.dot(v.astype(bf), w2_ref[...].astype(bf),
                      preferred_element_type=jnp.float32)
              + b2_ref[...])

    s3 = h2.reshape(_B, _N, _CH)
    t1 = jnp.transpose(s3, (1, 0, 2))
    t2 = jnp.transpose(t1, (0, 2, 1))
    flat = t2.reshape(_N * _CH, _B).T

    acc = jnp.zeros((_B, _H1), jnp.float32)
    for k in range(_NK):
        pltpu.make_async_copy(
            wf1_hbm.at[k * _KC:(k + 1) * _KC, :],
            wf1_vmem.at[k], sems.at[k]).wait()
        acc = acc + jnp.dot(flat[:, k * _KC:(k + 1) * _KC], wf1_vmem[k],
                            preferred_element_type=jnp.float32)

    t = jax.nn.relu(acc + bf1_ref[...])
    t = jax.nn.relu(jnp.dot(t, wf2_ref[...],
                            preferred_element_type=jnp.float32) + bf2_ref[...])
    o_ref[...] = jax.nn.sigmoid(
        jnp.dot(t, wf3_ref[...], preferred_element_type=jnp.float32)
        + bf3_ref[...])


def kernel(x, a, W1, b1, W2, b2, Wf1, bf1, Wf2, bf2, Wf3, bf3):
    x2 = x.reshape(_B * _N, _F)
    b1t = jnp.tile(b1, _B).reshape(1, _B * _CH)

    out = pl.pallas_call(
        _net_kernel,
        in_specs=[
            pl.BlockSpec((_B * _N, _F), lambda: (0, 0)),
            pl.BlockSpec((_N, _N), lambda: (0, 0)),
            pl.BlockSpec((_F, _CH), lambda: (0, 0)),
            pl.BlockSpec((1, _B * _CH), lambda: (0, 0)),
            pl.BlockSpec((_CH, _CH), lambda: (0, 0)),
            pl.BlockSpec((1, _CH), lambda: (0, 0)),
            pl.BlockSpec(memory_space=pl.ANY),
            pl.BlockSpec((1, _H1), lambda: (0, 0)),
            pl.BlockSpec((_H1, _H2), lambda: (0, 0)),
            pl.BlockSpec((1, _H2), lambda: (0, 0)),
            pl.BlockSpec((_H2, 1), lambda: (0, 0)),
            pl.BlockSpec((1, 1), lambda: (0, 0)),
        ],
        out_specs=pl.BlockSpec((_B, 1), lambda: (0, 0)),
        out_shape=jax.ShapeDtypeStruct((_B, 1), jnp.float32),
        scratch_shapes=[
            pltpu.VMEM((_NK, _KC, _H1), jnp.float32),
            pltpu.SemaphoreType.DMA((_NK,)),
        ],
    )(x2, a, W1, b1t, W2, b2.reshape(1, _CH), Wf1,
      bf1.reshape(1, _H1), Wf2, bf2.reshape(1, _H2), Wf3, bf3.reshape(1, 1))
    return out
